# Initial kernel scaffold; baseline (speedup 1.0000x reference)
import jax
import jax.numpy as jnp
from jax.experimental import pallas as pl


def kernel(x, emb_table, W, b):
    raise NotImplementedError("write your pallas kernel here")


# speedup vs baseline: 131.8122x; 131.8122x over previous
import functools

import jax
import jax.numpy as jnp
from jax import lax
from jax.experimental import pallas as pl
from jax.experimental.pallas import tpu as pltpu
from jax.experimental.pallas import tpu_sc as plsc

VOCAB_N = 1000
VPAD = 1024
EMB_D = 128
BATCH_N = 16384
HIST = 200


def _p_table_body(emb_ref, w_ref, out_ref):
    out_ref[...] = lax.dot_general(
        emb_ref[...], w_ref[...],
        dimension_numbers=(((1,), (1,)), ((), ())),
        preferred_element_type=jnp.float32,
        precision=lax.Precision.HIGHEST,
    )


def _compute_p_table(emb_padded, w):
    return pl.pallas_call(
        _p_table_body,
        out_shape=jax.ShapeDtypeStruct((VPAD, 1), jnp.float32),
    )(emb_padded, w)


def _make_sc_kernel():
    info = plsc.get_sparse_core_info()
    nc, ns = info.num_cores, info.num_subcores
    nw = nc * ns
    rpw = BATCH_N // nw
    mesh = plsc.VectorSubcoreMesh(core_axis_name="c", subcore_axis_name="s")

    @functools.partial(
        pl.kernel,
        mesh=mesh,
        out_type=jax.ShapeDtypeStruct((BATCH_N,), jnp.float32),
        compiler_params=pltpu.CompilerParams(needs_layout_passes=False),
        scratch_types=[
            pltpu.VMEM((rpw * HIST,), jnp.int32),
            pltpu.VMEM((VPAD,), jnp.float32),
            pltpu.VMEM((rpw,), jnp.float32),
            pltpu.VMEM((16,), jnp.float32),
            pltpu.SemaphoreType.DMA,
        ],
    )
    def sc_main(x_hbm, p_hbm, b_hbm, out_hbm, x_v, p_v, out_v, b_v, sem):
        wid = lax.axis_index("s") * nc + lax.axis_index("c")
        rb = wid * rpw
        cp = pltpu.async_copy(x_hbm.at[pl.ds(rb * HIST, rpw * HIST)], x_v, sem)
        pltpu.sync_copy(p_hbm, p_v)
        pltpu.sync_copy(b_hbm, b_v)
        cp.wait()
        bias = b_v[...]
        lane = lax.broadcasted_iota(jnp.int32, (16,), 0)

        def g_body(g, _):
            base_v = (g * 16 + lane) * HIST

            def l_body(l, acc):
                xv = plsc.load_gather(x_v, [base_v + l])
                return acc + plsc.load_gather(p_v, [xv])

            acc = lax.fori_loop(0, HIST, l_body, jnp.zeros((16,), jnp.float32))
            z = acc + bias
            out_v[pl.ds(g * 16, 16)] = 1.0 / (1.0 + jnp.exp(-z))
            return 0

        lax.fori_loop(0, rpw // 16, g_body, 0)
        pltpu.sync_copy(out_v, out_hbm.at[pl.ds(rb, rpw)])

    return sc_main


def kernel(x, emb_table, W, b):
    emb_padded = jnp.pad(emb_table, ((0, VPAD - VOCAB_N), (0, 0)))
    p = _compute_p_table(emb_padded, W).reshape(VPAD)
    b16 = jnp.broadcast_to(b, (16,))
    x_flat = x.reshape(BATCH_N * HIST)
    out = _make_sc_kernel()(x_flat, p, b16)
    return out.reshape(BATCH_N, 1)


# speedup vs baseline: 193.4942x; 1.4680x over previous
import functools

import jax
import jax.numpy as jnp
from jax import lax
from jax.experimental import pallas as pl
from jax.experimental.pallas import tpu as pltpu
from jax.experimental.pallas import tpu_sc as plsc

VOCAB_N = 1000
VPAD = 1024
EMB_D = 128
BATCH_N = 16384
HIST = 200


def _p_table_body(emb_ref, w_ref, out_ref):
    out_ref[...] = lax.dot_general(
        emb_ref[...], w_ref[...],
        dimension_numbers=(((1,), (1,)), ((), ())),
        preferred_element_type=jnp.float32,
        precision=lax.Precision.HIGHEST,
    )


def _compute_p_table(emb_padded, w):
    return pl.pallas_call(
        _p_table_body,
        out_shape=jax.ShapeDtypeStruct((VPAD, 1), jnp.float32),
    )(emb_padded, w)


def _make_sc_kernel():
    info = plsc.get_sparse_core_info()
    nc, ns = info.num_cores, info.num_subcores
    nw = nc * ns
    rpw = BATCH_N // nw
    mesh = plsc.VectorSubcoreMesh(core_axis_name="c", subcore_axis_name="s")

    @functools.partial(
        pl.kernel,
        mesh=mesh,
        out_type=jax.ShapeDtypeStruct((BATCH_N,), jnp.float32),
        compiler_params=pltpu.CompilerParams(needs_layout_passes=False),
        scratch_types=[
            pltpu.VMEM((rpw * HIST,), jnp.int32),
            pltpu.VMEM((VPAD,), jnp.float32),
            pltpu.VMEM((rpw,), jnp.float32),
            pltpu.VMEM((16,), jnp.float32),
            pltpu.SemaphoreType.DMA,
        ],
    )
    def sc_main(x_hbm, p_hbm, b_hbm, out_hbm, x_v, p_v, out_v, b_v, sem):
        wid = lax.axis_index("s") * nc + lax.axis_index("c")
        rb = wid * rpw
        cp = pltpu.async_copy(x_hbm.at[pl.ds(rb * HIST, rpw * HIST)], x_v, sem)
        pltpu.sync_copy(p_hbm, p_v)
        pltpu.sync_copy(b_hbm, b_v)
        cp.wait()
        bias = b_v[...]
        lane = lax.broadcasted_iota(jnp.int32, (16,), 0)

        def g_body(g, _):
            base_v = (g * 16 + lane) * HIST

            def l_body(i, accs):
                accs = list(accs)
                l0 = i * 8
                for u in range(8):
                    xv = plsc.load_gather(x_v, [base_v + (l0 + u)])
                    pv = plsc.load_gather(p_v, [xv])
                    accs[u % 4] = accs[u % 4] + pv
                return tuple(accs)

            zero = jnp.zeros((16,), jnp.float32)
            a0, a1, a2, a3 = lax.fori_loop(
                0, HIST // 8, l_body, (zero, zero, zero, zero)
            )
            z = (a0 + a1) + (a2 + a3) + bias
            out_v[pl.ds(g * 16, 16)] = 1.0 / (1.0 + jnp.exp(-z))
            return 0

        lax.fori_loop(0, rpw // 16, g_body, 0)
        pltpu.sync_copy(out_v, out_hbm.at[pl.ds(rb, rpw)])

    return sc_main


def kernel(x, emb_table, W, b):
    emb_padded = jnp.pad(emb_table, ((0, VPAD - VOCAB_N), (0, 0)))
    p = _compute_p_table(emb_padded, W).reshape(VPAD)
    b16 = jnp.broadcast_to(b, (16,))
    x_flat = x.reshape(BATCH_N * HIST)
    out = _make_sc_kernel()(x_flat, p, b16)
    return out.reshape(BATCH_N, 1)
